# Initial kernel scaffold; baseline (speedup 1.0000x reference)
import jax
import jax.numpy as jnp
from jax.experimental import pallas as pl


def kernel(x, edge_index, edge_weight, stress, params):
    raise NotImplementedError("write your pallas kernel here")


# speedup vs baseline: 1.0469x; 1.0469x over previous
import functools

import jax
import jax.numpy as jnp
from jax import lax
from jax.experimental import pallas as pl
from jax.experimental.pallas import tpu as pltpu

N = 10000
E = 320000
D_IN = 128
D_H = 128
S_DIM = 16
S_EMB = 32
HEAD_H = 64
L = 3
GAMMA_MIN = 0.1
GAMMA_MAX = 2.0

BN = 1000
GRID = N // BN


def _prologue_body(x_ref, stress_ref, win_ref, bin_ref, wg_ref, bg_ref,
                   wb_ref, bb_ref, fg_ref, fgb_ref, fb_ref, fbb_ref,
                   h0_ref, g_ref, b_ref, gam3_ref, bet3_ref):
    x = x_ref[...]
    h0_ref[...] = jnp.dot(x, win_ref[...],
                          preferred_element_type=jnp.float32) + bin_ref[...]
    s = stress_ref[...]
    for l in range(L):
        pre = jnp.dot(s, wg_ref[l], preferred_element_type=jnp.float32) + bg_ref[l]
        gamma = GAMMA_MIN + (GAMMA_MAX - GAMMA_MIN) * jax.nn.sigmoid(pre)
        beta = jnp.dot(s, wb_ref[l], preferred_element_type=jnp.float32) + bb_ref[l]
        g_ref[l] = jnp.dot(gamma, fg_ref[l],
                           preferred_element_type=jnp.float32) + fgb_ref[l]
        b_ref[l] = jnp.dot(beta, fb_ref[l],
                           preferred_element_type=jnp.float32) + fbb_ref[l]
        if l == L - 1:
            gam3_ref[...] = gamma
            bet3_ref[...] = beta


def _prologue(x, stress, params):
    lps = params["layers"]
    Wg = jnp.stack([lp["Wg"] for lp in lps])
    bg = jnp.stack([lp["bg"] for lp in lps])[:, None, :]
    Wb = jnp.stack([lp["Wb"] for lp in lps])
    bb = jnp.stack([lp["bb"] for lp in lps])[:, None, :]
    Fg = jnp.stack([lp["Fg"] for lp in lps])
    Fgb = jnp.stack([lp["Fg_b"] for lp in lps])[:, None, :]
    Fb = jnp.stack([lp["Fb"] for lp in lps])
    Fbb = jnp.stack([lp["Fb_b"] for lp in lps])[:, None, :]
    full = lambda shp: pl.BlockSpec(shp, lambda i: (0,) * len(shp))
    row = lambda d: pl.BlockSpec((BN, d), lambda i: (i, 0))
    lrow = lambda d: pl.BlockSpec((L, BN, d), lambda i: (0, i, 0))
    return pl.pallas_call(
        _prologue_body,
        grid=(GRID,),
        in_specs=[
            row(D_IN), row(S_DIM),
            full((D_IN, D_H)), full((1, D_H)),
            full((L, S_DIM, S_EMB)), full((L, 1, S_EMB)),
            full((L, S_DIM, S_EMB)), full((L, 1, S_EMB)),
            full((L, S_EMB, D_H)), full((L, 1, D_H)),
            full((L, S_EMB, D_H)), full((L, 1, D_H)),
        ],
        out_specs=[row(D_H), lrow(D_H), lrow(D_H), row(S_EMB), row(S_EMB)],
        out_shape=[
            jax.ShapeDtypeStruct((N, D_H), jnp.float32),
            jax.ShapeDtypeStruct((L, N, D_H), jnp.float32),
            jax.ShapeDtypeStruct((L, N, D_H), jnp.float32),
            jax.ShapeDtypeStruct((N, S_EMB), jnp.float32),
            jax.ShapeDtypeStruct((N, S_EMB), jnp.float32),
        ],
    )(x, stress, params["W_in"], params["b_in"][None, :],
      Wg, bg, Wb, bb, Fg, Fgb, Fb, Fbb)


def _layer_body(h_ref, a0_ref, a1_ref, ws_ref, bs_ref, wn_ref, g_ref, b_ref,
                out_ref):
    h = h_ref[...]
    agg = a0_ref[...] + a1_ref[...]
    z = (jnp.dot(h, ws_ref[...], preferred_element_type=jnp.float32)
         + bs_ref[...]
         + jnp.dot(agg, wn_ref[...], preferred_element_type=jnp.float32))
    out_ref[...] = jnp.maximum(g_ref[...] * z + b_ref[...], 0.0) + h


def _layer_update(h, a0, a1, lp, g, b):
    full = lambda shp: pl.BlockSpec(shp, lambda i: (0,) * len(shp))
    row = lambda d: pl.BlockSpec((BN, d), lambda i: (i, 0))
    return pl.pallas_call(
        _layer_body,
        grid=(GRID,),
        in_specs=[row(D_H), row(D_H), row(D_H),
                  full((D_H, D_H)), full((1, D_H)), full((D_H, D_H)),
                  row(D_H), row(D_H)],
        out_specs=row(D_H),
        out_shape=jax.ShapeDtypeStruct((N, D_H), jnp.float32),
    )(h, a0, a1, lp["W_self"], lp["b_self"][None, :], lp["W_nbr"], g, b)


def _head_body(h_ref, gam_ref, bet_ref, w1a_ref, w1g_ref, w1b_ref, b1_ref,
               wo_ref, bo_ref, w2a_ref, w2g_ref, w2b_ref, b2_ref,
               wrec_ref, brec_ref, logit_ref, recon_ref):
    h = h_ref[...]
    gam = gam_ref[...]
    bet = bet_ref[...]
    dot = lambda a, w: jnp.dot(a, w, preferred_element_type=jnp.float32)
    h1 = jnp.maximum(dot(h, w1a_ref[...]) + dot(gam, w1g_ref[...])
                     + dot(bet, w1b_ref[...]) + b1_ref[...], 0.0)
    logit_ref[...] = (jnp.sum(h1 * wo_ref[...], axis=1, keepdims=True)
                      + bo_ref[...])
    h2 = jnp.maximum(dot(h, w2a_ref[...]) + dot(gam, w2g_ref[...])
                     + dot(bet, w2b_ref[...]) + b2_ref[...], 0.0)
    recon_ref[...] = dot(h2, wrec_ref[...]) + brec_ref[...]


def _head(h, gam3, bet3, hp):
    full = lambda shp: pl.BlockSpec(shp, lambda i: (0,) * len(shp))
    row = lambda d: pl.BlockSpec((BN, d), lambda i: (i, 0))
    W1, W2 = hp["W1"], hp["W2"]
    logits2, recon = pl.pallas_call(
        _head_body,
        grid=(GRID,),
        in_specs=[row(D_H), row(S_EMB), row(S_EMB),
                  full((D_H, HEAD_H)), full((S_EMB, HEAD_H)),
                  full((S_EMB, HEAD_H)), full((1, HEAD_H)),
                  full((1, HEAD_H)), full((1, 1)),
                  full((D_H, HEAD_H)), full((S_EMB, HEAD_H)),
                  full((S_EMB, HEAD_H)), full((1, HEAD_H)),
                  full((HEAD_H, D_IN)), full((1, D_IN))],
        out_specs=[row(1), row(D_IN)],
        out_shape=[jax.ShapeDtypeStruct((N, 1), jnp.float32),
                   jax.ShapeDtypeStruct((N, D_IN), jnp.float32)],
    )(h, gam3, bet3,
      W1[:D_H], W1[D_H:D_H + S_EMB], W1[D_H + S_EMB:], hp["b1"][None, :],
      hp["w_out"][:, 0][None, :], hp["b_out"][None, :],
      W2[:D_H], W2[D_H:D_H + S_EMB], W2[D_H + S_EMB:], hp["b2"][None, :],
      hp["W_rec"], hp["b_rec"][None, :])
    return logits2[:, 0], recon


def _segment_sum(h, src, dst, w):
    msg = h[src] * w[:, None]
    agg = jax.ops.segment_sum(msg, dst, num_segments=N)
    return agg, jnp.zeros_like(agg)


def kernel(x, edge_index, edge_weight, stress, params):
    src = edge_index[0]
    dst = edge_index[1]
    h, gstack, bstack, gam3, bet3 = _prologue(x, stress, params)
    for l, lp in enumerate(params["layers"]):
        a0, a1 = _segment_sum(h, src, dst, edge_weight)
        h = _layer_update(h, a0, a1, lp, gstack[l], bstack[l])
    logits, recon = _head(h, gam3, bet3, params["head"])
    return logits, recon, h


# speedup vs baseline: 3.4599x; 3.3048x over previous
import functools

import jax
import jax.numpy as jnp
from jax import lax
from jax.experimental import pallas as pl
from jax.experimental.pallas import tpu as pltpu
from jax.experimental.pallas import tpu_sc as plsc

N = 10000
E = 320000
D_IN = 128
D_H = 128
S_DIM = 16
S_EMB = 32
HEAD_H = 64
L = 3
GAMMA_MIN = 0.1
GAMMA_MAX = 2.0

BN = 1000
GRID = N // BN


def _prologue_body(x_ref, stress_ref, win_ref, bin_ref, wg_ref, bg_ref,
                   wb_ref, bb_ref, fg_ref, fgb_ref, fb_ref, fbb_ref,
                   h0_ref, g_ref, b_ref, gam3_ref, bet3_ref):
    x = x_ref[...]
    h0_ref[...] = jnp.dot(x, win_ref[...],
                          preferred_element_type=jnp.float32) + bin_ref[...]
    s = stress_ref[...]
    for l in range(L):
        pre = jnp.dot(s, wg_ref[l], preferred_element_type=jnp.float32) + bg_ref[l]
        gamma = GAMMA_MIN + (GAMMA_MAX - GAMMA_MIN) * jax.nn.sigmoid(pre)
        beta = jnp.dot(s, wb_ref[l], preferred_element_type=jnp.float32) + bb_ref[l]
        g_ref[l] = jnp.dot(gamma, fg_ref[l],
                           preferred_element_type=jnp.float32) + fgb_ref[l]
        b_ref[l] = jnp.dot(beta, fb_ref[l],
                           preferred_element_type=jnp.float32) + fbb_ref[l]
        if l == L - 1:
            gam3_ref[...] = gamma
            bet3_ref[...] = beta


def _prologue(x, stress, params):
    lps = params["layers"]
    Wg = jnp.stack([lp["Wg"] for lp in lps])
    bg = jnp.stack([lp["bg"] for lp in lps])[:, None, :]
    Wb = jnp.stack([lp["Wb"] for lp in lps])
    bb = jnp.stack([lp["bb"] for lp in lps])[:, None, :]
    Fg = jnp.stack([lp["Fg"] for lp in lps])
    Fgb = jnp.stack([lp["Fg_b"] for lp in lps])[:, None, :]
    Fb = jnp.stack([lp["Fb"] for lp in lps])
    Fbb = jnp.stack([lp["Fb_b"] for lp in lps])[:, None, :]
    full = lambda shp: pl.BlockSpec(shp, lambda i: (0,) * len(shp))
    row = lambda d: pl.BlockSpec((BN, d), lambda i: (i, 0))
    lrow = lambda d: pl.BlockSpec((L, BN, d), lambda i: (0, i, 0))
    return pl.pallas_call(
        _prologue_body,
        grid=(GRID,),
        in_specs=[
            row(D_IN), row(S_DIM),
            full((D_IN, D_H)), full((1, D_H)),
            full((L, S_DIM, S_EMB)), full((L, 1, S_EMB)),
            full((L, S_DIM, S_EMB)), full((L, 1, S_EMB)),
            full((L, S_EMB, D_H)), full((L, 1, D_H)),
            full((L, S_EMB, D_H)), full((L, 1, D_H)),
        ],
        out_specs=[row(D_H), lrow(D_H), lrow(D_H), row(S_EMB), row(S_EMB)],
        out_shape=[
            jax.ShapeDtypeStruct((N, D_H), jnp.float32),
            jax.ShapeDtypeStruct((L, N, D_H), jnp.float32),
            jax.ShapeDtypeStruct((L, N, D_H), jnp.float32),
            jax.ShapeDtypeStruct((N, S_EMB), jnp.float32),
            jax.ShapeDtypeStruct((N, S_EMB), jnp.float32),
        ],
    )(x, stress, params["W_in"], params["b_in"][None, :],
      Wg, bg, Wb, bb, Fg, Fgb, Fb, Fbb)


def _layer_body(h_ref, a0_ref, a1_ref, ws_ref, bs_ref, wn_ref, g_ref, b_ref,
                out_ref):
    h = h_ref[...]
    agg = a0_ref[...] + a1_ref[...]
    z = (jnp.dot(h, ws_ref[...], preferred_element_type=jnp.float32)
         + bs_ref[...]
         + jnp.dot(agg, wn_ref[...], preferred_element_type=jnp.float32))
    out_ref[...] = jnp.maximum(g_ref[...] * z + b_ref[...], 0.0) + h


def _layer_update(h, a0, a1, lp, g, b):
    full = lambda shp: pl.BlockSpec(shp, lambda i: (0,) * len(shp))
    row = lambda d: pl.BlockSpec((BN, d), lambda i: (i, 0))
    return pl.pallas_call(
        _layer_body,
        grid=(GRID,),
        in_specs=[row(D_H), row(D_H), row(D_H),
                  full((D_H, D_H)), full((1, D_H)), full((D_H, D_H)),
                  row(D_H), row(D_H)],
        out_specs=row(D_H),
        out_shape=jax.ShapeDtypeStruct((N, D_H), jnp.float32),
    )(h, a0, a1, lp["W_self"], lp["b_self"][None, :], lp["W_nbr"], g, b)


def _head_body(h_ref, gam_ref, bet_ref, w1a_ref, w1g_ref, w1b_ref, b1_ref,
               wo_ref, bo_ref, w2a_ref, w2g_ref, w2b_ref, b2_ref,
               wrec_ref, brec_ref, logit_ref, recon_ref):
    h = h_ref[...]
    gam = gam_ref[...]
    bet = bet_ref[...]
    dot = lambda a, w: jnp.dot(a, w, preferred_element_type=jnp.float32)
    h1 = jnp.maximum(dot(h, w1a_ref[...]) + dot(gam, w1g_ref[...])
                     + dot(bet, w1b_ref[...]) + b1_ref[...], 0.0)
    logit_ref[...] = (jnp.sum(h1 * wo_ref[...], axis=1, keepdims=True)
                      + bo_ref[...])
    h2 = jnp.maximum(dot(h, w2a_ref[...]) + dot(gam, w2g_ref[...])
                     + dot(bet, w2b_ref[...]) + b2_ref[...], 0.0)
    recon_ref[...] = dot(h2, wrec_ref[...]) + brec_ref[...]


def _head(h, gam3, bet3, hp):
    full = lambda shp: pl.BlockSpec(shp, lambda i: (0,) * len(shp))
    row = lambda d: pl.BlockSpec((BN, d), lambda i: (i, 0))
    W1, W2 = hp["W1"], hp["W2"]
    logits2, recon = pl.pallas_call(
        _head_body,
        grid=(GRID,),
        in_specs=[row(D_H), row(S_EMB), row(S_EMB),
                  full((D_H, HEAD_H)), full((S_EMB, HEAD_H)),
                  full((S_EMB, HEAD_H)), full((1, HEAD_H)),
                  full((1, HEAD_H)), full((1, 1)),
                  full((D_H, HEAD_H)), full((S_EMB, HEAD_H)),
                  full((S_EMB, HEAD_H)), full((1, HEAD_H)),
                  full((HEAD_H, D_IN)), full((1, D_IN))],
        out_specs=[row(1), row(D_IN)],
        out_shape=[jax.ShapeDtypeStruct((N, 1), jnp.float32),
                   jax.ShapeDtypeStruct((N, D_IN), jnp.float32)],
    )(h, gam3, bet3,
      W1[:D_H], W1[D_H:D_H + S_EMB], W1[D_H + S_EMB:], hp["b1"][None, :],
      hp["w_out"][:, 0][None, :], hp["b_out"][None, :],
      W2[:D_H], W2[D_H:D_H + S_EMB], W2[D_H + S_EMB:], hp["b2"][None, :],
      hp["W_rec"], hp["b_rec"][None, :])
    return logits2[:, 0], recon


NC = 2
NS = 16
NW = NC * NS
EW = E // NW
C = 80
NCHUNK = EW // C
RPT = (N // NS) // 8 * 8
REM = N - RPT * NS


def _sc_segsum_body(h_hbm, src_hbm, dst_hbm, w_hbm, zeros_hbm, out_hbm,
                    src_c, dst_c, w_c, rows_v, agg_sh):
    c = lax.axis_index("c")
    s = lax.axis_index("s")
    wid = c * NS + s
    pltpu.sync_copy(zeros_hbm.at[pl.ds(s * RPT, RPT)],
                    agg_sh.at[pl.ds(s * RPT, RPT)])

    @pl.when(s == 0)
    def _zero_rem():
        pltpu.sync_copy(zeros_hbm.at[pl.ds(RPT * NS, REM)],
                        agg_sh.at[pl.ds(RPT * NS, REM)])

    plsc.subcore_barrier()

    def chunk(j, carry):
        pltpu.sync_copy(src_hbm.at[wid, j], src_c)
        pltpu.sync_copy(dst_hbm.at[wid, j], dst_c)
        pltpu.sync_copy(w_hbm.at[wid, j], w_c)
        pltpu.sync_copy(h_hbm.at[src_c], rows_v)
        for g in range(C // 16):
            wvec = w_c[pl.ds(g * 16, 16)]
            for i in range(16):
                we = wvec[i]
                e = g * 16 + i
                for d in range(D_H // 16):
                    sl = pl.ds(d * 16, 16)
                    rows_v[e, sl] = rows_v[e, sl] * we
        pltpu.sync_copy(rows_v, agg_sh.at[dst_c], add=True)
        return carry

    lax.fori_loop(0, NCHUNK, chunk, 0)
    plsc.subcore_barrier()
    pltpu.sync_copy(agg_sh.at[pl.ds(s * RPT, RPT)],
                    out_hbm.at[c, pl.ds(s * RPT, RPT)])

    @pl.when(s == 0)
    def _write_rem():
        pltpu.sync_copy(agg_sh.at[pl.ds(RPT * NS, REM)],
                        out_hbm.at[c, pl.ds(RPT * NS, REM)])


def _segment_sum(h, src, dst, w):
    f = pl.kernel(
        _sc_segsum_body,
        out_type=jax.ShapeDtypeStruct((NC, N, D_H), jnp.float32),
        mesh=plsc.VectorSubcoreMesh(core_axis_name="c", subcore_axis_name="s"),
        scratch_types=[
            pltpu.VMEM((C,), jnp.int32),
            pltpu.VMEM((C,), jnp.int32),
            pltpu.VMEM((C,), jnp.float32),
            pltpu.VMEM((C, D_H), jnp.float32),
            pltpu.VMEM_SHARED((N, D_H), jnp.float32),
        ],
    )
    out = f(h, src.reshape(NW, NCHUNK, C), dst.reshape(NW, NCHUNK, C),
            w.reshape(NW, NCHUNK, C), jnp.zeros((N, D_H), jnp.float32))
    return out[0], out[1]


def kernel(x, edge_index, edge_weight, stress, params):
    src = edge_index[0]
    dst = edge_index[1]
    h, gstack, bstack, gam3, bet3 = _prologue(x, stress, params)
    for l, lp in enumerate(params["layers"]):
        a0, a1 = _segment_sum(h, src, dst, edge_weight)
        h = _layer_update(h, a0, a1, lp, gstack[l], bstack[l])
    logits, recon = _head(h, gam3, bet3, params["head"])
    return logits, recon, h


# speedup vs baseline: 7.5019x; 2.1682x over previous
import functools

import jax
import jax.numpy as jnp
from jax import lax
from jax.experimental import pallas as pl
from jax.experimental.pallas import tpu as pltpu
from jax.experimental.pallas import tpu_sc as plsc

N = 10000
E = 320000
D_IN = 128
D_H = 128
S_DIM = 16
S_EMB = 32
HEAD_H = 64
L = 3
GAMMA_MIN = 0.1
GAMMA_MAX = 2.0

BN = 1000
GRID = N // BN


def _prologue_body(x_ref, stress_ref, win_ref, bin_ref, wg_ref, bg_ref,
                   wb_ref, bb_ref, fg_ref, fgb_ref, fb_ref, fbb_ref,
                   h0_ref, g_ref, b_ref, gam3_ref, bet3_ref):
    x = x_ref[...]
    h0_ref[...] = jnp.dot(x, win_ref[...],
                          preferred_element_type=jnp.float32) + bin_ref[...]
    s = stress_ref[...]
    for l in range(L):
        pre = jnp.dot(s, wg_ref[l], preferred_element_type=jnp.float32) + bg_ref[l]
        gamma = GAMMA_MIN + (GAMMA_MAX - GAMMA_MIN) * jax.nn.sigmoid(pre)
        beta = jnp.dot(s, wb_ref[l], preferred_element_type=jnp.float32) + bb_ref[l]
        g_ref[l] = jnp.dot(gamma, fg_ref[l],
                           preferred_element_type=jnp.float32) + fgb_ref[l]
        b_ref[l] = jnp.dot(beta, fb_ref[l],
                           preferred_element_type=jnp.float32) + fbb_ref[l]
        if l == L - 1:
            gam3_ref[...] = gamma
            bet3_ref[...] = beta


def _prologue(x, stress, params):
    lps = params["layers"]
    Wg = jnp.stack([lp["Wg"] for lp in lps])
    bg = jnp.stack([lp["bg"] for lp in lps])[:, None, :]
    Wb = jnp.stack([lp["Wb"] for lp in lps])
    bb = jnp.stack([lp["bb"] for lp in lps])[:, None, :]
    Fg = jnp.stack([lp["Fg"] for lp in lps])
    Fgb = jnp.stack([lp["Fg_b"] for lp in lps])[:, None, :]
    Fb = jnp.stack([lp["Fb"] for lp in lps])
    Fbb = jnp.stack([lp["Fb_b"] for lp in lps])[:, None, :]
    full = lambda shp: pl.BlockSpec(shp, lambda i: (0,) * len(shp))
    row = lambda d: pl.BlockSpec((BN, d), lambda i: (i, 0))
    lrow = lambda d: pl.BlockSpec((L, BN, d), lambda i: (0, i, 0))
    return pl.pallas_call(
        _prologue_body,
        grid=(GRID,),
        in_specs=[
            row(D_IN), row(S_DIM),
            full((D_IN, D_H)), full((1, D_H)),
            full((L, S_DIM, S_EMB)), full((L, 1, S_EMB)),
            full((L, S_DIM, S_EMB)), full((L, 1, S_EMB)),
            full((L, S_EMB, D_H)), full((L, 1, D_H)),
            full((L, S_EMB, D_H)), full((L, 1, D_H)),
        ],
        out_specs=[row(D_H), lrow(D_H), lrow(D_H), row(S_EMB), row(S_EMB)],
        out_shape=[
            jax.ShapeDtypeStruct((N, D_H), jnp.float32),
            jax.ShapeDtypeStruct((L, N, D_H), jnp.float32),
            jax.ShapeDtypeStruct((L, N, D_H), jnp.float32),
            jax.ShapeDtypeStruct((N, S_EMB), jnp.float32),
            jax.ShapeDtypeStruct((N, S_EMB), jnp.float32),
        ],
    )(x, stress, params["W_in"], params["b_in"][None, :],
      Wg, bg, Wb, bb, Fg, Fgb, Fb, Fbb)


def _layer_body(h_ref, a0_ref, a1_ref, ws_ref, bs_ref, wn_ref, g_ref, b_ref,
                out_ref):
    h = h_ref[...]
    agg = a0_ref[...] + a1_ref[...]
    z = (jnp.dot(h, ws_ref[...], preferred_element_type=jnp.float32)
         + bs_ref[...]
         + jnp.dot(agg, wn_ref[...], preferred_element_type=jnp.float32))
    out_ref[...] = jnp.maximum(g_ref[...] * z + b_ref[...], 0.0) + h


def _layer_update(h, a0, a1, lp, g, b):
    full = lambda shp: pl.BlockSpec(shp, lambda i: (0,) * len(shp))
    row = lambda d: pl.BlockSpec((BN, d), lambda i: (i, 0))
    return pl.pallas_call(
        _layer_body,
        grid=(GRID,),
        in_specs=[row(D_H), row(D_H), row(D_H),
                  full((D_H, D_H)), full((1, D_H)), full((D_H, D_H)),
                  row(D_H), row(D_H)],
        out_specs=row(D_H),
        out_shape=jax.ShapeDtypeStruct((N, D_H), jnp.float32),
    )(h, a0, a1, lp["W_self"], lp["b_self"][None, :], lp["W_nbr"], g, b)


def _head_body(h_ref, gam_ref, bet_ref, w1a_ref, w1g_ref, w1b_ref, b1_ref,
               wo_ref, bo_ref, w2a_ref, w2g_ref, w2b_ref, b2_ref,
               wrec_ref, brec_ref, logit_ref, recon_ref):
    h = h_ref[...]
    gam = gam_ref[...]
    bet = bet_ref[...]
    dot = lambda a, w: jnp.dot(a, w, preferred_element_type=jnp.float32)
    h1 = jnp.maximum(dot(h, w1a_ref[...]) + dot(gam, w1g_ref[...])
                     + dot(bet, w1b_ref[...]) + b1_ref[...], 0.0)
    logit_ref[...] = (jnp.sum(h1 * wo_ref[...], axis=1, keepdims=True)
                      + bo_ref[...])
    h2 = jnp.maximum(dot(h, w2a_ref[...]) + dot(gam, w2g_ref[...])
                     + dot(bet, w2b_ref[...]) + b2_ref[...], 0.0)
    recon_ref[...] = dot(h2, wrec_ref[...]) + brec_ref[...]


def _head(h, gam3, bet3, hp):
    full = lambda shp: pl.BlockSpec(shp, lambda i: (0,) * len(shp))
    row = lambda d: pl.BlockSpec((BN, d), lambda i: (i, 0))
    W1, W2 = hp["W1"], hp["W2"]
    logits2, recon = pl.pallas_call(
        _head_body,
        grid=(GRID,),
        in_specs=[row(D_H), row(S_EMB), row(S_EMB),
                  full((D_H, HEAD_H)), full((S_EMB, HEAD_H)),
                  full((S_EMB, HEAD_H)), full((1, HEAD_H)),
                  full((1, HEAD_H)), full((1, 1)),
                  full((D_H, HEAD_H)), full((S_EMB, HEAD_H)),
                  full((S_EMB, HEAD_H)), full((1, HEAD_H)),
                  full((HEAD_H, D_IN)), full((1, D_IN))],
        out_specs=[row(1), row(D_IN)],
        out_shape=[jax.ShapeDtypeStruct((N, 1), jnp.float32),
                   jax.ShapeDtypeStruct((N, D_IN), jnp.float32)],
    )(h, gam3, bet3,
      W1[:D_H], W1[D_H:D_H + S_EMB], W1[D_H + S_EMB:], hp["b1"][None, :],
      hp["w_out"][:, 0][None, :], hp["b_out"][None, :],
      W2[:D_H], W2[D_H:D_H + S_EMB], W2[D_H + S_EMB:], hp["b2"][None, :],
      hp["W_rec"], hp["b_rec"][None, :])
    return logits2[:, 0], recon


NC = 2
NS = 16
NW = NC * NS
EW = E // NW
C = 80
NCHUNK = 128
EPAD = NW * NCHUNK * C
RPT = (N // NS) // 8 * 8
REM = N - RPT * NS


def _sc_segsum_body(h_hbm, idx_hbm, zeros_hbm, out_hbm,
                    ib, rows, dstb, agg_sh,
                    isem0, isem1, gsem0, gsem1, ssem0, ssem1):
    c = lax.axis_index("c")
    s = lax.axis_index("s")
    wid = c * NS + s
    isem = (isem0, isem1)
    gsem = (gsem0, gsem1)
    ssem = (ssem0, ssem1)

    pltpu.sync_copy(zeros_hbm.at[pl.ds(s * RPT, RPT)],
                    agg_sh.at[pl.ds(s * RPT, RPT)])

    @pl.when(s == 0)
    def _zero_rem():
        pltpu.sync_copy(zeros_hbm.at[pl.ds(RPT * NS, REM)],
                        agg_sh.at[pl.ds(RPT * NS, REM)])

    plsc.subcore_barrier()

    def fetch_idx(j, b):
        pltpu.async_copy(idx_hbm.at[wid, j], ib.at[b], isem[b])

    def wait_idx(b):
        pltpu.make_async_copy(idx_hbm.at[wid, 0], ib.at[b], isem[b]).wait()

    def issue_gather(jb, b):
        pltpu.async_copy(h_hbm.at[ib.at[jb, 0]], rows.at[b], gsem[b])

    def wait_gather(b):
        pltpu.make_async_copy(h_hbm.at[ib.at[b, 0]], rows.at[b],
                              gsem[b]).wait()

    def issue_scatter(b):
        pltpu.async_copy(rows.at[b], agg_sh.at[dstb.at[b]], ssem[b],
                         add=True)

    def wait_scatter(b):
        pltpu.make_async_copy(rows.at[b], agg_sh.at[dstb.at[b]],
                              ssem[b]).wait()

    def process_chunk(j, b, first, do_next_gather, do_fetch):
        nb = 1 - b
        if not first:
            wait_scatter(nb)
        if do_next_gather:
            wait_idx(nb)
            issue_gather(nb, nb)
        wait_gather(b)
        for g in range(C // 16):
            sl = pl.ds(g * 16, 16)
            dstb[b, sl] = ib[b, 1, sl]
            wvec = lax.bitcast_convert_type(ib[b, 2, sl], jnp.float32)
            for i in range(16):
                we = wvec[i]
                e = g * 16 + i
                for d in range(D_H // 16):
                    dsl = pl.ds(d * 16, 16)
                    rows[b, e, dsl] = rows[b, e, dsl] * we
        issue_scatter(b)
        if do_fetch:
            fetch_idx(j + 2, b)

    fetch_idx(0, 0)
    fetch_idx(1, 1)
    wait_idx(0)
    issue_gather(0, 0)

    def two_chunks(i, carry):
        j = 2 * i
        process_chunk(j, 0, False, True, True)
        process_chunk(j + 1, 1, False, True, True)
        return carry

    process_chunk(0, 0, True, True, True)
    process_chunk(1, 1, False, True, True)
    lax.fori_loop(1, (NCHUNK - 2) // 2, two_chunks, 0)
    process_chunk(NCHUNK - 2, 0, False, True, False)
    process_chunk(NCHUNK - 1, 1, False, False, False)
    wait_scatter(1)

    plsc.subcore_barrier()
    pltpu.sync_copy(agg_sh.at[pl.ds(s * RPT, RPT)],
                    out_hbm.at[c, pl.ds(s * RPT, RPT)])

    @pl.when(s == 0)
    def _write_rem():
        pltpu.sync_copy(agg_sh.at[pl.ds(RPT * NS, REM)],
                        out_hbm.at[c, pl.ds(RPT * NS, REM)])


def _pack_edges(src, dst, w):
    pad = EPAD - E
    pe = (jnp.arange(pad, dtype=jnp.int32) * 37) % N
    srcp = jnp.concatenate([src, pe]).reshape(NW, NCHUNK, C)
    dstp = jnp.concatenate([dst, pe]).reshape(NW, NCHUNK, C)
    wp = jnp.concatenate([w, jnp.zeros((pad,), jnp.float32)])
    wbits = lax.bitcast_convert_type(wp, jnp.int32).reshape(NW, NCHUNK, C)
    return jnp.stack([srcp, dstp, wbits], axis=2)


def _segment_sum(h, idx_packed):
    f = pl.kernel(
        _sc_segsum_body,
        out_type=jax.ShapeDtypeStruct((NC, N, D_H), jnp.float32),
        mesh=plsc.VectorSubcoreMesh(core_axis_name="c", subcore_axis_name="s"),
        scratch_types=[
            pltpu.VMEM((2, 3, C), jnp.int32),
            pltpu.VMEM((2, C, D_H), jnp.float32),
            pltpu.VMEM((2, C), jnp.int32),
            pltpu.VMEM_SHARED((N, D_H), jnp.float32),
            pltpu.SemaphoreType.DMA,
            pltpu.SemaphoreType.DMA,
            pltpu.SemaphoreType.DMA,
            pltpu.SemaphoreType.DMA,
            pltpu.SemaphoreType.DMA,
            pltpu.SemaphoreType.DMA,
        ],
    )
    out = f(h, idx_packed, jnp.zeros((N, D_H), jnp.float32))
    return out[0], out[1]


def kernel(x, edge_index, edge_weight, stress, params):
    idx_packed = _pack_edges(edge_index[0], edge_index[1], edge_weight)
    h, gstack, bstack, gam3, bet3 = _prologue(x, stress, params)
    for l, lp in enumerate(params["layers"]):
        a0, a1 = _segment_sum(h, idx_packed)
        h = _layer_update(h, a0, a1, lp, gstack[l], bstack[l])
    logits, recon = _head(h, gam3, bet3, params["head"])
    return logits, recon, h
